# Initial kernel scaffold; baseline (speedup 1.0000x reference)
import jax
import jax.numpy as jnp
from jax.experimental import pallas as pl


def kernel(x, edge_index, batch_index, W0, b0, W1, b1, W2, b2, W3, b3, Wout, bout):
    raise NotImplementedError("write your pallas kernel here")


# speedup vs baseline: 33.5024x; 33.5024x over previous
import functools

import jax
import jax.numpy as jnp
from jax import lax
from jax.experimental import pallas as pl
from jax.experimental.pallas import tpu as pltpu
from jax.experimental.pallas import tpu_sc as plsc

N = 100000
E = 3200000
G = 512
H = 16

NC = 2
NS = 16
NW = NC * NS

N_PAD = 100096
NPW = N_PAD // NW
NPT = N_PAD // NS
DUMP = N

CPW = 784
UNROLL = 8
T_OUT = CPW // UNROLL
E_PAD = NW * CPW * 128
ER = E_PAD // 128

G_PAD = 520
DUMP_G = G

_mesh = plsc.VectorSubcoreMesh(core_axis_name="c", subcore_axis_name="s")
_sc_params = pltpu.CompilerParams(use_tc_tiling_on_sc=False)
_pool_params = pltpu.CompilerParams(use_tc_tiling_on_sc=False,
                                    needs_layout_passes=False)


def _worker_ids():
    cid = lax.axis_index("c")
    sid = lax.axis_index("s")
    return cid, sid, cid * NS + sid


@functools.partial(
    pl.kernel,
    out_type=jax.ShapeDtypeStruct((NC, N_PAD, H), jnp.float32),
    mesh=_mesh,
    compiler_params=_sc_params,
    scratch_types=[
        pltpu.VMEM((UNROLL, 128), jnp.int32),
        pltpu.VMEM((128, H), jnp.float32),
        pltpu.VMEM_SHARED((N_PAD, H), jnp.float32),
    ],
)
def _deg_sc(dst_hbm, zeros_hbm, ones_hbm, out_hbm, didx_v, ones_v, acc_sh):
    cid, sid, wid = _worker_ids()
    pltpu.sync_copy(ones_hbm, ones_v)
    pltpu.sync_copy(zeros_hbm.at[pl.ds(sid * NPT, NPT)],
                    acc_sh.at[pl.ds(sid * NPT, NPT)])
    plsc.subcore_barrier()

    def body(t, _):
        row0 = wid * CPW + t * UNROLL
        pltpu.sync_copy(dst_hbm.at[pl.ds(row0, UNROLL)], didx_v)
        for j in range(UNROLL):
            pltpu.sync_copy(ones_v, acc_sh.at[didx_v.at[j]], add=True)
        return 0

    lax.fori_loop(0, T_OUT, body, 0)
    plsc.subcore_barrier()
    pltpu.sync_copy(acc_sh.at[pl.ds(sid * NPT, NPT)],
                    out_hbm.at[cid, pl.ds(sid * NPT, NPT)])


@functools.partial(
    pl.kernel,
    out_type=jax.ShapeDtypeStruct((NC, N_PAD, H), jnp.float32),
    mesh=_mesh,
    compiler_params=_sc_params,
    scratch_types=[
        pltpu.VMEM((UNROLL, 128), jnp.int32),
        pltpu.VMEM((UNROLL, 128), jnp.int32),
        pltpu.VMEM((UNROLL * 128, H), jnp.float32),
        pltpu.SemaphoreType.DMA,
        pltpu.VMEM_SHARED((N_PAD, H), jnp.float32),
    ],
)
def _agg_sc(u_hbm, src_hbm, dst_hbm, zeros_hbm, out_hbm,
            sidx_v, didx_v, rows_v, sem, acc_sh):
    cid, sid, wid = _worker_ids()
    pltpu.sync_copy(zeros_hbm.at[pl.ds(sid * NPT, NPT)],
                    acc_sh.at[pl.ds(sid * NPT, NPT)])
    plsc.subcore_barrier()

    def body(t, _):
        row0 = wid * CPW + t * UNROLL
        pltpu.sync_copy(src_hbm.at[pl.ds(row0, UNROLL)], sidx_v)
        pltpu.sync_copy(dst_hbm.at[pl.ds(row0, UNROLL)], didx_v)
        descs = []
        for j in range(UNROLL):
            descs.append(
                pltpu.async_copy(u_hbm.at[sidx_v.at[j]],
                                 rows_v.at[pl.ds(j * 128, 128)], sem))
        for d in descs:
            d.wait()
        for j in range(UNROLL):
            pltpu.sync_copy(rows_v.at[pl.ds(j * 128, 128)],
                            acc_sh.at[didx_v.at[j]], add=True)
        return 0

    lax.fori_loop(0, T_OUT, body, 0)
    plsc.subcore_barrier()
    pltpu.sync_copy(acc_sh.at[pl.ds(sid * NPT, NPT)],
                    out_hbm.at[cid, pl.ds(sid * NPT, NPT)])


@functools.partial(
    pl.kernel,
    out_type=(
        jax.ShapeDtypeStruct((NW, G_PAD, H), jnp.float32),
        jax.ShapeDtypeStruct((NW, G_PAD, H), jnp.float32),
        jax.ShapeDtypeStruct((NW, G_PAD, H), jnp.float32),
    ),
    mesh=_mesh,
    compiler_params=_pool_params,
    scratch_types=[
        pltpu.VMEM((NPW, H), jnp.float32),
        pltpu.VMEM((NPW,), jnp.int32),
        pltpu.VMEM((G_PAD, H), jnp.float32),
        pltpu.VMEM((G_PAD, H), jnp.float32),
        pltpu.VMEM((G_PAD, H), jnp.float32),
    ],
)
def _pool_sc(z_hbm, batch_hbm, omax_hbm, osum_hbm, ocnt_hbm,
             z_v, b_v, amax, asum, acnt):
    cid, sid, wid = _worker_ids()
    pltpu.sync_copy(z_hbm.at[pl.ds(wid * NPW, NPW)], z_v)
    pltpu.sync_copy(batch_hbm.at[pl.ds(wid * NPW, NPW)], b_v)

    col = lax.iota(jnp.int32, 16)
    ones = jnp.full((16,), 1.0, jnp.float32)
    neg = jnp.full((16,), -2.0, jnp.float32)
    zero = jnp.full((16,), 0.0, jnp.float32)

    def init(g, _):
        gv = jnp.full((16,), g, jnp.int32)
        plsc.store_scatter(amax, [gv, col], neg)
        plsc.store_scatter(asum, [gv, col], zero)
        plsc.store_scatter(acnt, [gv, col], zero)
        return 0

    lax.fori_loop(0, G_PAD, init, 0)

    def body(i, _):
        iv = jnp.full((16,), i, jnp.int32)
        bv = plsc.load_gather(b_v, [iv])
        row = plsc.load_gather(z_v, [iv, col])
        cur = plsc.load_gather(amax, [bv, col])
        plsc.store_scatter(amax, [bv, col], jnp.maximum(cur, row))
        plsc.addupdate_scatter(asum, [bv, col], row)
        plsc.addupdate_scatter(acnt, [bv, col], ones)
        return 0

    lax.fori_loop(0, NPW, body, 0)
    pltpu.sync_copy(amax, omax_hbm.at[wid])
    pltpu.sync_copy(asum, osum_hbm.at[wid])
    pltpu.sync_copy(acnt, ocnt_hbm.at[wid])


BLK = 3128
GRID = N_PAD // BLK

_row_spec = pl.BlockSpec((BLK, H), lambda i: (i, 0))
_w_spec = pl.BlockSpec((H, H), lambda i: (0, 0))
_b_spec = pl.BlockSpec((1, H), lambda i: (0, 0))


def _first_tc_body(d0_ref, d1_ref, x_ref, w_ref, s_ref, u_ref):
    deg = d0_ref[:, 0:1] + d1_ref[:, 0:1] + 1.0
    s = lax.rsqrt(deg)
    s_ref[...] = jnp.broadcast_to(s, (BLK, H))
    u_ref[...] = s * jnp.dot(x_ref[...], w_ref[...],
                             preferred_element_type=jnp.float32)


def _first_tc(d0, d1, x_pad, w0):
    return pl.pallas_call(
        _first_tc_body,
        grid=(GRID,),
        in_specs=[_row_spec, _row_spec, _row_spec, _w_spec],
        out_specs=(_row_spec, _row_spec),
        out_shape=(
            jax.ShapeDtypeStruct((N_PAD, H), jnp.float32),
            jax.ShapeDtypeStruct((N_PAD, H), jnp.float32),
        ),
    )(d0, d1, x_pad, w0)


def _mid_tc_body(e0_ref, e1_ref, u_ref, s_ref, w_ref, b_ref, o_ref):
    s = s_ref[...]
    z = jnp.tanh(s * (e0_ref[...] + e1_ref[...] + u_ref[...]) + b_ref[...])
    o_ref[...] = s * jnp.dot(z, w_ref[...], preferred_element_type=jnp.float32)


def _mid_tc(e0, e1, u, s, w, b):
    return pl.pallas_call(
        _mid_tc_body,
        grid=(GRID,),
        in_specs=[_row_spec, _row_spec, _row_spec, _row_spec, _w_spec, _b_spec],
        out_specs=_row_spec,
        out_shape=jax.ShapeDtypeStruct((N_PAD, H), jnp.float32),
    )(e0, e1, u, s, w, b)


def _last_tc_body(e0_ref, e1_ref, u_ref, s_ref, b_ref, o_ref):
    o_ref[...] = jnp.tanh(
        s_ref[...] * (e0_ref[...] + e1_ref[...] + u_ref[...]) + b_ref[...])


def _last_tc(e0, e1, u, s, b):
    return pl.pallas_call(
        _last_tc_body,
        grid=(GRID,),
        in_specs=[_row_spec, _row_spec, _row_spec, _row_spec, _b_spec],
        out_specs=_row_spec,
        out_shape=jax.ShapeDtypeStruct((N_PAD, H), jnp.float32),
    )(e0, e1, u, s, b)


def _final_tc_body(pm_ref, ps_ref, pc_ref, wo_ref, bo_ref, o_ref):
    gmax = jnp.max(pm_ref[...], axis=0)[:G]
    gsum = jnp.sum(ps_ref[...], axis=0)[:G]
    cnt = jnp.sum(pc_ref[...], axis=0)[:G, 0:1]
    gmean = gsum / jnp.maximum(cnt, 1.0)
    wo = wo_ref[...]
    o_ref[...] = (jnp.dot(gmax, wo[:H], preferred_element_type=jnp.float32)
                  + jnp.dot(gmean, wo[H:], preferred_element_type=jnp.float32)
                  + bo_ref[...])


def _final_tc(pmax, psum, pcnt, wout, bout):
    return pl.pallas_call(
        _final_tc_body,
        out_shape=jax.ShapeDtypeStruct((G, 1), jnp.float32),
    )(pmax, psum, pcnt, wout, bout.reshape(1, 1))


def kernel(x, edge_index, batch_index, W0, b0, W1, b1, W2, b2, W3, b3,
           Wout, bout):
    f32 = jnp.float32
    src = edge_index[0].astype(jnp.int32)
    dst = edge_index[1].astype(jnp.int32)
    pad_e = jnp.full((E_PAD - E,), DUMP, jnp.int32)
    src2d = jnp.concatenate([src, pad_e]).reshape(ER, 128)
    dst2d = jnp.concatenate([dst, pad_e]).reshape(ER, 128)

    x_pad = jnp.zeros((N_PAD, H), f32).at[:N, :x.shape[1]].set(x.astype(f32))
    w0p = jnp.zeros((H, H), f32).at[:W0.shape[0]].set(W0.astype(f32))

    batch_pad = jnp.concatenate([
        batch_index.astype(jnp.int32),
        jnp.full((N_PAD - N,), DUMP_G, jnp.int32)])

    zeros_hbm = jnp.zeros((N_PAD, H), f32)
    ones_hbm = jnp.ones((128, H), f32)

    dparts = _deg_sc(dst2d, zeros_hbm, ones_hbm)
    s_arr, u = _first_tc(dparts[0], dparts[1], x_pad, w0p)

    for w, b in ((W1, b0), (W2, b1), (W3, b2)):
        e = _agg_sc(u, src2d, dst2d, zeros_hbm)
        u = _mid_tc(e[0], e[1], u, s_arr, w.astype(f32),
                    b.astype(f32).reshape(1, H))

    e = _agg_sc(u, src2d, dst2d, zeros_hbm)
    z4 = _last_tc(e[0], e[1], u, s_arr, b3.astype(f32).reshape(1, H))

    pmax, psum, pcnt = _pool_sc(z4, batch_pad)
    return _final_tc(pmax, psum, pcnt, Wout.astype(f32), bout.astype(f32))
